# baseline (speedup 1.0000x reference)
import functools

import jax
import jax.numpy as jnp
from jax import lax
from jax.experimental import pallas as pl
from jax.experimental.pallas import tpu as pltpu
from jax.experimental.pallas import tpu_sc as plsc

_NUM_EMB = 8192
_EMB_DIM = 256
_N = 16384
_COM_COST = 0.25

_BN = 128
_NB = _N // _BN

_BN2 = 1024
_NB2 = _N // _BN2

_GW = 128


_WSPLIT = 4096


def _tc1_body(x_ref, wt_ref, xsq_ref, wsq_ref, enc_ref, idx_ref):
    x = x_ref[...]
    mm = lax.dot_general(x, wt_ref[...], (((1,), (0,)), ((), ())),
                         preferred_element_type=jnp.float32)
    t1 = xsq_ref[...] + wsq_ref[...]
    d = t1 - 2.0 * mm
    kio = lax.broadcasted_iota(jnp.int32, (_BN, _NUM_EMB), 1)

    def win(a, b):
        dw = d[:, a:b]
        kw = kio[:, a:b]
        m = jnp.min(dw, axis=1, keepdims=True)
        i = jnp.min(jnp.where(dw == m, kw, _NUM_EMB), axis=1)
        return m[:, 0], i

    m0, i0 = win(0, _WSPLIT)
    m1, i1 = win(_WSPLIT, _NUM_EMB)
    b0 = m0.astype(jnp.bfloat16).astype(jnp.float32)
    idx = jnp.where(m1 < b0, i1, i0)

    enc_ref[...] = jnp.where(kio == idx[:, None],
                             jnp.float32(1.0), jnp.float32(0.0))
    idx_ref[0, 0, :] = idx


def _tc1(inputs, w_t, xsq, wsq_row):
    enc, idx3 = pl.pallas_call(
        _tc1_body,
        grid=(_NB,),
        in_specs=[
            pl.BlockSpec((_BN, _EMB_DIM), lambda i: (i, 0)),
            pl.BlockSpec((_EMB_DIM, _NUM_EMB), lambda i: (0, 0)),
            pl.BlockSpec((_BN, 1), lambda i: (i, 0)),
            pl.BlockSpec((1, _NUM_EMB), lambda i: (0, 0)),
        ],
        out_specs=[
            pl.BlockSpec((_BN, _NUM_EMB), lambda i: (i, 0)),
            pl.BlockSpec((1, 1, _BN), lambda i: (i, 0, 0)),
        ],
        out_shape=[
            jax.ShapeDtypeStruct((_N, _NUM_EMB), jnp.float32),
            jax.ShapeDtypeStruct((_NB, 1, _BN), jnp.int32),
        ],
    )(inputs, w_t, xsq, wsq_row)
    return enc, idx3.reshape(_N)


def _sc_gather(w, idx):
    idx2 = idx.reshape(1, _N)
    mesh = plsc.VectorSubcoreMesh(core_axis_name="core",
                                  subcore_axis_name="subcore")

    @functools.partial(
        pl.kernel,
        out_type=jax.ShapeDtypeStruct((_N, _EMB_DIM), jnp.float32),
        mesh=mesh,
    )
    def k(w_hbm, i_hbm, o_hbm):
        def body(i_vmem, o_vmem):
            pltpu.sync_copy(w_hbm.at[i_vmem.at[0]], o_vmem)

        pltpu.emit_pipeline(
            body,
            grid=(_N // _GW,),
            in_specs=[pl.BlockSpec((1, _GW), index_map=lambda i: (0, i))],
            out_specs=[pl.BlockSpec((_GW, _EMB_DIM), index_map=lambda i: (i, 0))],
            core_axis_name=("core", "subcore"),
            dimension_semantics=(pltpu.PARALLEL,),
        )(i_hbm, o_hbm)

    return k(w, idx2)


def _tc2_body(x_ref, q_ref, qst_ref, loss_ref):
    i = pl.program_id(0)
    x = x_ref[...]
    q = q_ref[...]
    dqx = q - x
    qst_ref[...] = x + dqx

    @pl.when(i == 0)
    def _():
        loss_ref[...] = jnp.zeros_like(loss_ref)

    loss_ref[...] += jnp.sum(dqx * dqx)

    @pl.when(i == _NB2 - 1)
    def _():
        scale = jnp.float32((1.0 + _COM_COST) / (_N * _EMB_DIM))
        loss_ref[...] = loss_ref[...] * scale


def _tc2(inputs, quantized):
    qst, loss11 = pl.pallas_call(
        _tc2_body,
        grid=(_NB2,),
        in_specs=[
            pl.BlockSpec((_BN2, _EMB_DIM), lambda i: (i, 0)),
            pl.BlockSpec((_BN2, _EMB_DIM), lambda i: (i, 0)),
        ],
        out_specs=[
            pl.BlockSpec((_BN2, _EMB_DIM), lambda i: (i, 0)),
            pl.BlockSpec((1, 1), lambda i: (0, 0)),
        ],
        out_shape=[
            jax.ShapeDtypeStruct((_N, _EMB_DIM), jnp.float32),
            jax.ShapeDtypeStruct((1, 1), jnp.float32),
        ],
    )(inputs, quantized)
    return qst, loss11.reshape(())


def kernel(inputs, W):
    xsq = jnp.sum(inputs ** 2, axis=1, keepdims=True)
    wsq = jnp.sum(W ** 2, axis=1)
    w_t = W.T
    enc, idx = _tc1(inputs, w_t, xsq, wsq.reshape(1, _NUM_EMB))
    quantized = _sc_gather(W, idx)
    qst, loss = _tc2(inputs, quantized)
    return loss, qst, enc


# speedup vs baseline: 1.2049x; 1.2049x over previous
import functools

import jax
import jax.numpy as jnp
from jax import lax
from jax.experimental import pallas as pl
from jax.experimental.pallas import tpu as pltpu
from jax.experimental.pallas import tpu_sc as plsc

_NUM_EMB = 8192
_EMB_DIM = 256
_N = 16384
_COM_COST = 0.25

_BN = 256
_NB = _N // _BN

_BN2 = 1024
_NB2 = _N // _BN2

_GW = 128


_WSPLIT = 4096


def _tc1_body(x_ref, wt_ref, xsq_ref, wsq_ref, kiof_ref, enc_ref, idx_ref):
    x = x_ref[...]
    mm = lax.dot_general(x, wt_ref[...], (((1,), (0,)), ((), ())),
                         preferred_element_type=jnp.float32)
    t1 = xsq_ref[...] + wsq_ref[...]
    d = t1 - 2.0 * mm
    kiof = kiof_ref[...]

    def win(a, b):
        dw = d[:, a:b]
        kw = kiof[:, a:b]
        m = jnp.min(dw, axis=1, keepdims=True)
        i = jnp.min(jnp.where(dw == m, kw, jnp.float32(_NUM_EMB)), axis=1)
        return m[:, 0], i

    m0, i0 = win(0, _WSPLIT)
    m1, i1 = win(_WSPLIT, _NUM_EMB)
    b0 = m0.astype(jnp.bfloat16).astype(jnp.float32)
    idxf = jnp.where(m1 < b0, i1, i0)

    enc_ref[...] = jnp.where(kiof == idxf[:, None],
                             jnp.float32(1.0), jnp.float32(0.0))
    idx_ref[0, 0, :] = idxf.astype(jnp.int32)


def _tc1(inputs, w_t, xsq, wsq_row):
    kiof = lax.broadcasted_iota(jnp.float32, (1, _NUM_EMB), 1)
    enc, idx3 = pl.pallas_call(
        _tc1_body,
        grid=(_NB,),
        in_specs=[
            pl.BlockSpec((_BN, _EMB_DIM), lambda i: (i, 0)),
            pl.BlockSpec((_EMB_DIM, _NUM_EMB), lambda i: (0, 0)),
            pl.BlockSpec((_BN, 1), lambda i: (i, 0)),
            pl.BlockSpec((1, _NUM_EMB), lambda i: (0, 0)),
            pl.BlockSpec((1, _NUM_EMB), lambda i: (0, 0)),
        ],
        out_specs=[
            pl.BlockSpec((_BN, _NUM_EMB), lambda i: (i, 0)),
            pl.BlockSpec((1, 1, _BN), lambda i: (i, 0, 0)),
        ],
        out_shape=[
            jax.ShapeDtypeStruct((_N, _NUM_EMB), jnp.float32),
            jax.ShapeDtypeStruct((_NB, 1, _BN), jnp.int32),
        ],
    )(inputs, w_t, xsq, wsq_row, kiof)
    return enc, idx3.reshape(_N)


def _sc_gather(w, idx):
    idx2 = idx.reshape(1, _N)
    mesh = plsc.VectorSubcoreMesh(core_axis_name="core",
                                  subcore_axis_name="subcore")

    @functools.partial(
        pl.kernel,
        out_type=jax.ShapeDtypeStruct((_N, _EMB_DIM), jnp.float32),
        mesh=mesh,
    )
    def k(w_hbm, i_hbm, o_hbm):
        def body(i_vmem, o_vmem):
            pltpu.sync_copy(w_hbm.at[i_vmem.at[0]], o_vmem)

        pltpu.emit_pipeline(
            body,
            grid=(_N // _GW,),
            in_specs=[pl.BlockSpec((1, _GW), index_map=lambda i: (0, i))],
            out_specs=[pl.BlockSpec((_GW, _EMB_DIM), index_map=lambda i: (i, 0))],
            core_axis_name=("core", "subcore"),
            dimension_semantics=(pltpu.PARALLEL,),
        )(i_hbm, o_hbm)

    return k(w, idx2)


def _tc2_body(x_ref, q_ref, qst_ref, loss_ref):
    i = pl.program_id(0)
    x = x_ref[...]
    q = q_ref[...]
    dqx = q - x
    qst_ref[...] = x + dqx

    @pl.when(i == 0)
    def _():
        loss_ref[...] = jnp.zeros_like(loss_ref)

    loss_ref[...] += jnp.sum(dqx * dqx)

    @pl.when(i == _NB2 - 1)
    def _():
        scale = jnp.float32((1.0 + _COM_COST) / (_N * _EMB_DIM))
        loss_ref[...] = loss_ref[...] * scale


def _tc2(inputs, quantized):
    qst, loss11 = pl.pallas_call(
        _tc2_body,
        grid=(_NB2,),
        in_specs=[
            pl.BlockSpec((_BN2, _EMB_DIM), lambda i: (i, 0)),
            pl.BlockSpec((_BN2, _EMB_DIM), lambda i: (i, 0)),
        ],
        out_specs=[
            pl.BlockSpec((_BN2, _EMB_DIM), lambda i: (i, 0)),
            pl.BlockSpec((1, 1), lambda i: (0, 0)),
        ],
        out_shape=[
            jax.ShapeDtypeStruct((_N, _EMB_DIM), jnp.float32),
            jax.ShapeDtypeStruct((1, 1), jnp.float32),
        ],
    )(inputs, quantized)
    return qst, loss11.reshape(())


def kernel(inputs, W):
    xsq = jnp.sum(inputs ** 2, axis=1, keepdims=True)
    wsq = jnp.sum(W ** 2, axis=1)
    w_t = W.T
    enc, idx = _tc1(inputs, w_t, xsq, wsq.reshape(1, _NUM_EMB))
    quantized = _sc_gather(W, idx)
    qst, loss = _tc2(inputs, quantized)
    return loss, qst, enc


# speedup vs baseline: 1.2588x; 1.0447x over previous
import functools

import jax
import jax.numpy as jnp
from jax import lax
from jax.experimental import pallas as pl
from jax.experimental.pallas import tpu as pltpu
from jax.experimental.pallas import tpu_sc as plsc

_NUM_EMB = 8192
_EMB_DIM = 256
_N = 16384
_COM_COST = 0.25

_BN = 512
_NB = _N // _BN

_BN2 = 1024
_NB2 = _N // _BN2

_GW = 128


_WSPLIT = 4096


def _tc1_body(x_ref, wt_ref, xsq_ref, wsq_ref, kiof_ref, enc_ref, idx_ref):
    x = x_ref[...]
    mm = lax.dot_general(x, wt_ref[...], (((1,), (0,)), ((), ())),
                         preferred_element_type=jnp.float32)
    t1 = xsq_ref[...] + wsq_ref[...]
    d = t1 - 2.0 * mm
    kiof = kiof_ref[...]

    def win(a, b):
        dw = d[:, a:b]
        kw = kiof[:, a:b]
        m = jnp.min(dw, axis=1, keepdims=True)
        i = jnp.min(jnp.where(dw == m, kw, jnp.float32(_NUM_EMB)), axis=1)
        return m[:, 0], i

    m0, i0 = win(0, _WSPLIT)
    m1, i1 = win(_WSPLIT, _NUM_EMB)
    b0 = m0.astype(jnp.bfloat16).astype(jnp.float32)
    idxf = jnp.where(m1 < b0, i1, i0)

    enc_ref[...] = jnp.where(kiof == idxf[:, None],
                             jnp.float32(1.0), jnp.float32(0.0))
    idx_ref[0, 0, :] = idxf.astype(jnp.int32)


def _tc1(inputs, w_t, xsq, wsq_row):
    kiof = lax.broadcasted_iota(jnp.float32, (1, _NUM_EMB), 1)
    enc, idx3 = pl.pallas_call(
        _tc1_body,
        grid=(_NB,),
        in_specs=[
            pl.BlockSpec((_BN, _EMB_DIM), lambda i: (i, 0)),
            pl.BlockSpec((_EMB_DIM, _NUM_EMB), lambda i: (0, 0)),
            pl.BlockSpec((_BN, 1), lambda i: (i, 0)),
            pl.BlockSpec((1, _NUM_EMB), lambda i: (0, 0)),
            pl.BlockSpec((1, _NUM_EMB), lambda i: (0, 0)),
        ],
        out_specs=[
            pl.BlockSpec((_BN, _NUM_EMB), lambda i: (i, 0)),
            pl.BlockSpec((1, 1, _BN), lambda i: (i, 0, 0)),
        ],
        out_shape=[
            jax.ShapeDtypeStruct((_N, _NUM_EMB), jnp.float32),
            jax.ShapeDtypeStruct((_NB, 1, _BN), jnp.int32),
        ],
    )(inputs, w_t, xsq, wsq_row, kiof)
    return enc, idx3.reshape(_N)


def _sc_gather(w, idx):
    idx2 = idx.reshape(1, _N)
    mesh = plsc.VectorSubcoreMesh(core_axis_name="core",
                                  subcore_axis_name="subcore")

    @functools.partial(
        pl.kernel,
        out_type=jax.ShapeDtypeStruct((_N, _EMB_DIM), jnp.float32),
        mesh=mesh,
    )
    def k(w_hbm, i_hbm, o_hbm):
        def body(i_vmem, o_vmem):
            pltpu.sync_copy(w_hbm.at[i_vmem.at[0]], o_vmem)

        pltpu.emit_pipeline(
            body,
            grid=(_N // _GW,),
            in_specs=[pl.BlockSpec((1, _GW), index_map=lambda i: (0, i))],
            out_specs=[pl.BlockSpec((_GW, _EMB_DIM), index_map=lambda i: (i, 0))],
            core_axis_name=("core", "subcore"),
            dimension_semantics=(pltpu.PARALLEL,),
        )(i_hbm, o_hbm)

    return k(w, idx2)


def _tc2_body(x_ref, q_ref, qst_ref, loss_ref):
    i = pl.program_id(0)
    x = x_ref[...]
    q = q_ref[...]
    dqx = q - x
    qst_ref[...] = x + dqx

    @pl.when(i == 0)
    def _():
        loss_ref[...] = jnp.zeros_like(loss_ref)

    loss_ref[...] += jnp.sum(dqx * dqx)

    @pl.when(i == _NB2 - 1)
    def _():
        scale = jnp.float32((1.0 + _COM_COST) / (_N * _EMB_DIM))
        loss_ref[...] = loss_ref[...] * scale


def _tc2(inputs, quantized):
    qst, loss11 = pl.pallas_call(
        _tc2_body,
        grid=(_NB2,),
        in_specs=[
            pl.BlockSpec((_BN2, _EMB_DIM), lambda i: (i, 0)),
            pl.BlockSpec((_BN2, _EMB_DIM), lambda i: (i, 0)),
        ],
        out_specs=[
            pl.BlockSpec((_BN2, _EMB_DIM), lambda i: (i, 0)),
            pl.BlockSpec((1, 1), lambda i: (0, 0)),
        ],
        out_shape=[
            jax.ShapeDtypeStruct((_N, _EMB_DIM), jnp.float32),
            jax.ShapeDtypeStruct((1, 1), jnp.float32),
        ],
    )(inputs, quantized)
    return qst, loss11.reshape(())


def kernel(inputs, W):
    xsq = jnp.sum(inputs ** 2, axis=1, keepdims=True)
    wsq = jnp.sum(W ** 2, axis=1)
    w_t = W.T
    enc, idx = _tc1(inputs, w_t, xsq, wsq.reshape(1, _NUM_EMB))
    quantized = _sc_gather(W, idx)
    qst, loss = _tc2(inputs, quantized)
    return loss, qst, enc
